# baseline (speedup 1.0000x reference)
import functools

import jax
import jax.numpy as jnp
from jax.experimental import pallas as pl
from jax.experimental.pallas import tpu as pltpu

_VMEM_LIMIT = 40 * 1024 * 1024


def _se_kernel(x_ref, w1_ref, w2_ref, o_ref, *, inv_l):
    xs = x_ref[...]
    pooled = jnp.sum(xs, axis=-1, dtype=jnp.float32) * inv_l
    h = jax.lax.dot_general(
        pooled, w1_ref[...], (((1,), (1,)), ((), ())),
        preferred_element_type=jnp.float32)
    h = jnp.maximum(h, 0.0)
    g = jax.lax.dot_general(
        h, w2_ref[...], (((1,), (1,)), ((), ())),
        preferred_element_type=jnp.float32)
    g = jax.nn.sigmoid(g)
    o_ref[...] = xs * g.astype(o_ref.dtype)[:, :, None]


@functools.partial(jax.jit, static_argnames=("tb",))
def _se_call(x, w1, w2, tb):
    B, C, L = x.shape
    Cr = w1.shape[0]
    body = functools.partial(_se_kernel, inv_l=1.0 / L)
    return pl.pallas_call(
        body,
        out_shape=jax.ShapeDtypeStruct((B, C, L), x.dtype),
        grid=(B // tb,),
        in_specs=[
            pl.BlockSpec((tb, C, L), lambda b: (b, 0, 0)),
            pl.BlockSpec((Cr, C), lambda b: (0, 0)),
            pl.BlockSpec((C, Cr), lambda b: (0, 0)),
        ],
        out_specs=pl.BlockSpec((tb, C, L), lambda b: (b, 0, 0)),
        compiler_params=pltpu.CompilerParams(
            dimension_semantics=("parallel",),
            vmem_limit_bytes=_VMEM_LIMIT,
        ),
    )(x, w1, w2)


def kernel(x, w1, w2):
    B, C, L = x.shape
    itemsize = jnp.dtype(x.dtype).itemsize
    tb = 1
    for d in range(B, 0, -1):
        if B % d == 0 and 8 * d * C * L * itemsize + 2**21 <= _VMEM_LIMIT:
            tb = d
            break
    return _se_call(x, w1, w2, tb)


# speedup vs baseline: 1.0328x; 1.0328x over previous
import functools

import jax
import jax.numpy as jnp
from jax.experimental import pallas as pl
from jax.experimental.pallas import tpu as pltpu

_VMEM_LIMIT = 58 * 1024 * 1024


def _excite(pooled, w1, w2):
    h = jax.lax.dot_general(pooled, w1, (((1,), (1,)), ((), ())),
                            preferred_element_type=jnp.float32)
    h = jnp.maximum(h, 0.0)
    g = jax.lax.dot_general(h, w2, (((1,), (1,)), ((), ())),
                            preferred_element_type=jnp.float32)
    return jax.nn.sigmoid(g)


def _se_ring_kernel(x_hbm, w1_ref, w2_ref, o_hbm, b0, b1, b2, rsem, wsem,
                    *, n_chunks, chunk, inv_l):
    core = pl.program_id(0)
    bufs = (b0, b1, b2)

    def rd(j):
        base = (core * n_chunks + j) * chunk
        pltpu.make_async_copy(
            x_hbm.at[pl.ds(base, chunk)], bufs[j % 3], rsem.at[j % 3]).start()

    def rd_wait(j):
        pltpu.make_async_copy(
            bufs[j % 3], bufs[j % 3], rsem.at[j % 3]).wait()

    def wr(j):
        base = (core * n_chunks + j) * chunk
        pltpu.make_async_copy(
            bufs[j % 3], o_hbm.at[pl.ds(base, chunk)], wsem.at[j % 3]).start()

    def wr_wait(j):
        pltpu.make_async_copy(
            bufs[j % 3], bufs[j % 3], wsem.at[j % 3]).wait()

    rd(0)
    for j in range(n_chunks):
        if j >= 2:
            wr_wait(j - 2)
        if j + 1 < n_chunks:
            rd(j + 1)
        rd_wait(j)
        xb = bufs[j % 3][...]
        pooled = jnp.sum(xb, axis=-1, dtype=jnp.float32) * inv_l
        g = _excite(pooled, w1_ref[...], w2_ref[...])
        bufs[j % 3][...] = xb * g.astype(xb.dtype)[:, :, None]
        wr(j)
    for j in range(max(n_chunks - 2, 0), n_chunks):
        wr_wait(j)


def _se_ring(x, w1, w2, n_chunks, chunk):
    B, C, L = x.shape
    Cr = w1.shape[0]
    body = functools.partial(_se_ring_kernel, n_chunks=n_chunks, chunk=chunk,
                             inv_l=1.0 / L)
    return pl.pallas_call(
        body,
        out_shape=jax.ShapeDtypeStruct((B, C, L), x.dtype),
        grid=(2,),
        in_specs=[
            pl.BlockSpec(memory_space=pl.ANY),
            pl.BlockSpec((Cr, C), lambda c: (0, 0)),
            pl.BlockSpec((C, Cr), lambda c: (0, 0)),
        ],
        out_specs=pl.BlockSpec(memory_space=pl.ANY),
        scratch_shapes=[
            pltpu.VMEM((chunk, C, L), x.dtype),
            pltpu.VMEM((chunk, C, L), x.dtype),
            pltpu.VMEM((chunk, C, L), x.dtype),
            pltpu.SemaphoreType.DMA((3,)),
            pltpu.SemaphoreType.DMA((3,)),
        ],
        compiler_params=pltpu.CompilerParams(
            dimension_semantics=("parallel",),
            vmem_limit_bytes=_VMEM_LIMIT,
        ),
    )(x, w1, w2)


def _se_fused_body(x_ref, w1_ref, w2_ref, o_ref, *, inv_l):
    xs = x_ref[...]
    pooled = jnp.sum(xs, axis=-1, dtype=jnp.float32) * inv_l
    g = _excite(pooled, w1_ref[...], w2_ref[...])
    o_ref[...] = xs * g.astype(o_ref.dtype)[:, :, None]


def _se_fused(x, w1, w2, tb):
    B, C, L = x.shape
    Cr = w1.shape[0]
    body = functools.partial(_se_fused_body, inv_l=1.0 / L)
    return pl.pallas_call(
        body,
        out_shape=jax.ShapeDtypeStruct((B, C, L), x.dtype),
        grid=(B // tb,),
        in_specs=[
            pl.BlockSpec((tb, C, L), lambda b: (b, 0, 0)),
            pl.BlockSpec((Cr, C), lambda b: (0, 0)),
            pl.BlockSpec((C, Cr), lambda b: (0, 0)),
        ],
        out_specs=pl.BlockSpec((tb, C, L), lambda b: (b, 0, 0)),
        compiler_params=pltpu.CompilerParams(
            dimension_semantics=("parallel",),
            vmem_limit_bytes=_VMEM_LIMIT,
        ),
    )(x, w1, w2)


@jax.jit
def _se(x, w1, w2):
    B, C, L = x.shape
    itemsize = jnp.dtype(x.dtype).itemsize
    row_bytes = C * L * itemsize
    chunk = 0
    for g in range(B // 2, 0, -1):
        if B % (2 * g) == 0 and 3 * g * row_bytes + 2**21 <= _VMEM_LIMIT \
                and B // (2 * g) >= 4:
            chunk = g
            break
    if chunk and L % 128 == 0 and C % 8 == 0:
        return _se_ring(x, w1, w2, B // (2 * chunk), chunk)
    for tb in range(B, 0, -1):
        if B % tb == 0 and 4 * tb * row_bytes + 2**21 <= _VMEM_LIMIT:
            return _se_fused(x, w1, w2, tb)
    return _se_fused(x, w1, w2, 1)


def kernel(x, w1, w2):
    return _se(x, w1, w2)
